# Initial kernel scaffold; baseline (speedup 1.0000x reference)
import jax
import jax.numpy as jnp
from jax.experimental import pallas as pl


def kernel(packed_data, batch_sizes, target, lengths):
    raise NotImplementedError("write your pallas kernel here")


# speedup vs baseline: 3.4733x; 3.4733x over previous
import functools

import jax
import jax.numpy as jnp
from jax import lax
from jax.experimental import pallas as pl
from jax.experimental.pallas import tpu as pltpu
from jax.experimental.pallas import tpu_sc as plsc

_TOTAL = 10880
_VOCAB = 2048
_B = 8
_TMAX = 2048
_ROWS = 320
_NBLK = _TOTAL // _ROWS
_L = 16
_NC = 2
_NS = 16


def _tc_body(x_ref, t_ref, o_ref):
    x = x_ref[...]
    m = jnp.max(x, axis=1)
    s = jnp.sum(jnp.exp(x - m[:, None]), axis=1)
    tgt = t_ref[0, 0, :]
    iota = lax.broadcasted_iota(jnp.int32, x.shape, 1)
    tv = jnp.sum(jnp.where(iota == tgt[:, None], x, 0.0), axis=1)
    o_ref[0, 0, :] = tv - m - jnp.log(s)


_tc_call = pl.pallas_call(
    _tc_body,
    grid=(_NBLK,),
    in_specs=[
        pl.BlockSpec((_ROWS, _VOCAB), lambda i: (i, 0)),
        pl.BlockSpec((1, 1, _ROWS), lambda i: (i, 0, 0)),
    ],
    out_specs=pl.BlockSpec((1, 1, _ROWS), lambda i: (i, 0, 0)),
    out_shape=jax.ShapeDtypeStruct((_NBLK, 1, _ROWS), jnp.float32),
)


def _sc_body(res_hbm, bs_hbm, len_hbm, out_hbm, res_v, bs_v, off_v, len_v,
             row_v):
    wid = lax.axis_index("s") * _NC + lax.axis_index("c")

    @pl.when(wid < _B)
    def _():
        b = wid
        pltpu.sync_copy(res_hbm, res_v)
        pltpu.sync_copy(bs_hbm, bs_v)
        pltpu.sync_copy(len_hbm, len_v)
        lane = lax.iota(jnp.int32, _L)
        bvec = jnp.full((_L,), b, jnp.int32)
        lbv = plsc.load_gather(len_v, [bvec])
        lb = jnp.max(lbv)

        def pre(i, running):
            t0 = i * _L
            bsv = bs_v[pl.ds(t0, _L)]
            inc = plsc.cumsum(bsv)
            off_v[pl.ds(t0, _L)] = running + inc - bsv
            return running + jnp.sum(bsv)

        lax.fori_loop(0, _TMAX // _L, pre, jnp.int32(0))

        def step(i, carry):
            se, spe = carry
            t0 = i * _L
            tvec = lane + t0
            valid = tvec < lb
            offs = off_v[pl.ds(t0, _L)]
            soffs = plsc.load_gather(off_v, [jnp.maximum(tvec - 1, 0)])
            idx = jnp.where(valid, offs + b, 0)
            sidx = jnp.where(valid, soffs + b, 0)
            padded = plsc.load_gather(res_v, [idx])
            shv = plsc.load_gather(res_v, [sidx])
            prop = jnp.where(tvec == 0, 1.0, jnp.exp(shv))
            e2 = jnp.where(valid, jnp.exp(prop), 0.0)
            return se + e2, spe + padded * e2

        z0 = jnp.zeros((_L,), jnp.float32)
        se, spe = lax.fori_loop(0, _TMAX // _L, step, (z0, z0))
        num = jnp.full((_L,), jnp.sum(spe), jnp.float32)
        den = jnp.full((_L,), jnp.sum(se), jnp.float32)
        row_v[...] = num * lbv.astype(jnp.float32) / den
        pltpu.sync_copy(row_v, out_hbm.at[b])


@functools.cache
def _sc_call():
    return pl.kernel(
        _sc_body,
        out_type=jax.ShapeDtypeStruct((_B, _L), jnp.float32),
        mesh=plsc.VectorSubcoreMesh(
            core_axis_name="c", subcore_axis_name="s",
            num_cores=_NC, num_subcores=_NS),
        scratch_types=[
            pltpu.VMEM((_TOTAL,), jnp.float32),
            pltpu.VMEM((_TMAX,), jnp.int32),
            pltpu.VMEM((_TMAX,), jnp.int32),
            pltpu.VMEM((_L,), jnp.int32),
            pltpu.VMEM((_L,), jnp.float32),
        ],
        compiler_params=pltpu.CompilerParams(needs_layout_passes=False),
    )


def kernel(packed_data, batch_sizes, target, lengths):
    target_r = target.astype(jnp.int32).reshape(_NBLK, 1, _ROWS)
    result = _tc_call(packed_data, target_r).reshape(_TOTAL)
    len_pad = jnp.zeros((_L,), jnp.int32).at[:_B].set(lengths.astype(jnp.int32))
    out = _sc_call()(result, batch_sizes.astype(jnp.int32), len_pad)
    return -jnp.sum(out[:, 0]) / _TOTAL


# speedup vs baseline: 3.9121x; 1.1263x over previous
import functools

import jax
import jax.numpy as jnp
from jax import lax
from jax.experimental import pallas as pl
from jax.experimental.pallas import tpu as pltpu
from jax.experimental.pallas import tpu_sc as plsc

_TOTAL = 10880
_VOCAB = 2048
_B = 8
_TMAX = 2048
_ROWS = 640
_NBLK = _TOTAL // _ROWS
_L = 16
_NC = 2
_NS = 16


def _tc_body(x_ref, t_ref, o_ref):
    x = x_ref[...]
    m = jnp.max(x, axis=1)
    s = jnp.sum(jnp.exp(x - m[:, None]), axis=1)
    tgt = t_ref[0, 0, :]
    iota = lax.broadcasted_iota(jnp.int32, x.shape, 1)
    tv = jnp.sum(jnp.where(iota == tgt[:, None], x, 0.0), axis=1)
    o_ref[0, 0, :] = tv - m - jnp.log(s)


_tc_call = pl.pallas_call(
    _tc_body,
    grid=(_NBLK,),
    in_specs=[
        pl.BlockSpec((_ROWS, _VOCAB), lambda i: (i, 0)),
        pl.BlockSpec((1, 1, _ROWS), lambda i: (i, 0, 0)),
    ],
    out_specs=pl.BlockSpec((1, 1, _ROWS), lambda i: (i, 0, 0)),
    out_shape=jax.ShapeDtypeStruct((_NBLK, 1, _ROWS), jnp.float32),
)


def _sc_body(res_hbm, bs_hbm, len_hbm, out_hbm, res_v, bs_v, off_v, len_v,
             row_v):
    wid = lax.axis_index("s") * _NC + lax.axis_index("c")

    @pl.when(wid < _B)
    def _():
        b = wid
        pltpu.sync_copy(res_hbm, res_v)
        pltpu.sync_copy(bs_hbm, bs_v)
        pltpu.sync_copy(len_hbm, len_v)
        lane = lax.iota(jnp.int32, _L)
        bvec = jnp.full((_L,), b, jnp.int32)
        lbv = plsc.load_gather(len_v, [bvec])
        lb = jnp.max(lbv)

        def pre(i, running):
            t0 = i * _L
            bsv = bs_v[pl.ds(t0, _L)]
            inc = plsc.cumsum(bsv)
            off_v[pl.ds(t0, _L)] = running + inc - bsv
            return running + jnp.sum(bsv)

        lax.fori_loop(0, _TMAX // _L, pre, jnp.int32(0))

        def step(i, carry):
            se, spe = carry
            t0 = i * _L
            tvec = lane + t0
            valid = tvec < lb
            offs = off_v[pl.ds(t0, _L)]
            soffs = plsc.load_gather(off_v, [jnp.maximum(tvec - 1, 0)])
            idx = jnp.where(valid, offs + b, 0)
            sidx = jnp.where(valid, soffs + b, 0)
            padded = plsc.load_gather(res_v, [idx])
            shv = plsc.load_gather(res_v, [sidx])
            prop = jnp.where(tvec == 0, 1.0, jnp.exp(shv))
            e2 = jnp.where(valid, jnp.exp(prop), 0.0)
            return se + e2, spe + padded * e2

        z0 = jnp.zeros((_L,), jnp.float32)
        se, spe = lax.fori_loop(0, _TMAX // _L, step, (z0, z0))
        num = jnp.full((_L,), jnp.sum(spe), jnp.float32)
        den = jnp.full((_L,), jnp.sum(se), jnp.float32)
        row_v[...] = num * lbv.astype(jnp.float32) / den
        pltpu.sync_copy(row_v, out_hbm.at[b])


@functools.cache
def _sc_call():
    return pl.kernel(
        _sc_body,
        out_type=jax.ShapeDtypeStruct((_B, _L), jnp.float32),
        mesh=plsc.VectorSubcoreMesh(
            core_axis_name="c", subcore_axis_name="s",
            num_cores=_NC, num_subcores=_NS),
        scratch_types=[
            pltpu.VMEM((_TOTAL,), jnp.float32),
            pltpu.VMEM((_TMAX,), jnp.int32),
            pltpu.VMEM((_TMAX,), jnp.int32),
            pltpu.VMEM((_L,), jnp.int32),
            pltpu.VMEM((_L,), jnp.float32),
        ],
        compiler_params=pltpu.CompilerParams(needs_layout_passes=False),
    )


def kernel(packed_data, batch_sizes, target, lengths):
    target_r = target.astype(jnp.int32).reshape(_NBLK, 1, _ROWS)
    result = _tc_call(packed_data, target_r).reshape(_TOTAL)
    len_pad = jnp.zeros((_L,), jnp.int32).at[:_B].set(lengths.astype(jnp.int32))
    out = _sc_call()(result, batch_sizes.astype(jnp.int32), len_pad)
    return -jnp.sum(out[:, 0]) / _TOTAL


# speedup vs baseline: 4.1262x; 1.0547x over previous
import functools

import jax
import jax.numpy as jnp
from jax import lax
from jax.experimental import pallas as pl
from jax.experimental.pallas import tpu as pltpu
from jax.experimental.pallas import tpu_sc as plsc

_TOTAL = 10880
_VOCAB = 2048
_B = 8
_TMAX = 2048
_ROWS = 1088
_NBLK = _TOTAL // _ROWS
_L = 16
_NC = 2
_NS = 16


def _tc_body(x_ref, t_ref, o_ref):
    x = x_ref[...]
    m = jnp.max(x, axis=1)
    s = jnp.sum(jnp.exp(x - m[:, None]), axis=1)
    tgt = t_ref[0, 0, :]
    iota = lax.broadcasted_iota(jnp.int32, x.shape, 1)
    tv = jnp.sum(jnp.where(iota == tgt[:, None], x, 0.0), axis=1)
    o_ref[0, 0, :] = tv - m - jnp.log(s)


_tc_call = pl.pallas_call(
    _tc_body,
    grid=(_NBLK,),
    in_specs=[
        pl.BlockSpec((_ROWS, _VOCAB), lambda i: (i, 0)),
        pl.BlockSpec((1, 1, _ROWS), lambda i: (i, 0, 0)),
    ],
    out_specs=pl.BlockSpec((1, 1, _ROWS), lambda i: (i, 0, 0)),
    out_shape=jax.ShapeDtypeStruct((_NBLK, 1, _ROWS), jnp.float32),
)


def _sc_body(res_hbm, bs_hbm, len_hbm, out_hbm, res_v, bs_v, off_v, len_v,
             row_v):
    wid = lax.axis_index("s") * _NC + lax.axis_index("c")

    @pl.when(wid < _B)
    def _():
        b = wid
        pltpu.sync_copy(res_hbm, res_v)
        pltpu.sync_copy(bs_hbm, bs_v)
        pltpu.sync_copy(len_hbm, len_v)
        lane = lax.iota(jnp.int32, _L)
        bvec = jnp.full((_L,), b, jnp.int32)
        lbv = plsc.load_gather(len_v, [bvec])
        lb = jnp.max(lbv)

        def pre(i, running):
            t0 = i * _L
            bsv = bs_v[pl.ds(t0, _L)]
            inc = plsc.cumsum(bsv)
            off_v[pl.ds(t0, _L)] = running + inc - bsv
            return running + jnp.sum(bsv)

        lax.fori_loop(0, _TMAX // _L, pre, jnp.int32(0))

        def step(i, carry):
            se, spe = carry
            t0 = i * _L
            tvec = lane + t0
            valid = tvec < lb
            offs = off_v[pl.ds(t0, _L)]
            soffs = plsc.load_gather(off_v, [jnp.maximum(tvec - 1, 0)])
            idx = jnp.where(valid, offs + b, 0)
            sidx = jnp.where(valid, soffs + b, 0)
            padded = plsc.load_gather(res_v, [idx])
            shv = plsc.load_gather(res_v, [sidx])
            prop = jnp.where(tvec == 0, 1.0, jnp.exp(shv))
            e2 = jnp.where(valid, jnp.exp(prop), 0.0)
            return se + e2, spe + padded * e2

        z0 = jnp.zeros((_L,), jnp.float32)
        se, spe = lax.fori_loop(0, _TMAX // _L, step, (z0, z0))
        num = jnp.full((_L,), jnp.sum(spe), jnp.float32)
        den = jnp.full((_L,), jnp.sum(se), jnp.float32)
        row_v[...] = num * lbv.astype(jnp.float32) / den
        pltpu.sync_copy(row_v, out_hbm.at[b])


@functools.cache
def _sc_call():
    return pl.kernel(
        _sc_body,
        out_type=jax.ShapeDtypeStruct((_B, _L), jnp.float32),
        mesh=plsc.VectorSubcoreMesh(
            core_axis_name="c", subcore_axis_name="s",
            num_cores=_NC, num_subcores=_NS),
        scratch_types=[
            pltpu.VMEM((_TOTAL,), jnp.float32),
            pltpu.VMEM((_TMAX,), jnp.int32),
            pltpu.VMEM((_TMAX,), jnp.int32),
            pltpu.VMEM((_L,), jnp.int32),
            pltpu.VMEM((_L,), jnp.float32),
        ],
        compiler_params=pltpu.CompilerParams(needs_layout_passes=False),
    )


def kernel(packed_data, batch_sizes, target, lengths):
    target_r = target.astype(jnp.int32).reshape(_NBLK, 1, _ROWS)
    result = _tc_call(packed_data, target_r).reshape(_TOTAL)
    len_pad = jnp.zeros((_L,), jnp.int32).at[:_B].set(lengths.astype(jnp.int32))
    out = _sc_call()(result, batch_sizes.astype(jnp.int32), len_pad)
    return -jnp.sum(out[:, 0]) / _TOTAL
